# Initial kernel scaffold; baseline (speedup 1.0000x reference)
import jax
import jax.numpy as jnp
from jax.experimental import pallas as pl


def kernel(x, idx_i, idx_j, Wq, bq, Wk, bk):
    raise NotImplementedError("write your pallas kernel here")


# speedup vs baseline: 2.4890x; 2.4890x over previous
import functools
import math

import jax
import jax.numpy as jnp
from jax import lax
from jax.experimental import pallas as pl
from jax.experimental.pallas import tpu as pltpu
from jax.experimental.pallas import tpu_sc as plsc

N, F, E = 10000, 256, 160000
M_TILE = 400
C = 128
NB = E // C
NC, NS, L = 2, 16, 16
NW = NC * NS
TPB = (NB + NW - 1) // NW


def _proj_kernel(x_ref, w_ref, b_ref, q_ref, k_ref, *, scale):
    res = jnp.dot(x_ref[...], w_ref[...], preferred_element_type=jnp.float32)
    res = res + b_ref[...]
    q_ref[...] = res[:, :F] * scale
    k_ref[...] = res[:, F:]


def _project(x, W, b, scale):
    return pl.pallas_call(
        functools.partial(_proj_kernel, scale=scale),
        grid=(N // M_TILE,),
        in_specs=[
            pl.BlockSpec((M_TILE, F), lambda i: (i, 0)),
            pl.BlockSpec((F, 2 * F), lambda i: (0, 0)),
            pl.BlockSpec((1, 2 * F), lambda i: (0, 0)),
        ],
        out_specs=[
            pl.BlockSpec((M_TILE, F), lambda i: (i, 0)),
            pl.BlockSpec((M_TILE, F), lambda i: (i, 0)),
        ],
        out_shape=[
            jax.ShapeDtypeStruct((N, F), jnp.float32),
            jax.ShapeDtypeStruct((N, F), jnp.float32),
        ],
    )(x, W, b)


def _sc_edge_dot(q, k, idx_i, idx_j):
    mesh = plsc.VectorSubcoreMesh(core_axis_name="c", subcore_axis_name="s")

    @functools.partial(
        pl.kernel,
        mesh=mesh,
        out_type=jax.ShapeDtypeStruct((E,), jnp.float32),
        scratch_types=[
            pltpu.VMEM((C,), jnp.int32),
            pltpu.VMEM((C,), jnp.int32),
            pltpu.VMEM((C, F), jnp.float32),
            pltpu.VMEM((C, F), jnp.float32),
            pltpu.VMEM((C,), jnp.float32),
            pltpu.VMEM((L * L,), jnp.float32),
            pltpu.SemaphoreType.DMA,
        ],
        compiler_params=pltpu.CompilerParams(needs_layout_passes=False),
    )
    def sc_kernel(q_hbm, k_hbm, ii_hbm, jj_hbm, out_hbm,
                  ii_v, jj_v, qrows, krows, out_v, accflat, sem):
        wid = lax.axis_index("s") * NC + lax.axis_index("c")

        def block_body(t, carry):
            bid = t * NW + wid

            @pl.when(bid < NB)
            def _():
                base = bid * C
                pltpu.sync_copy(ii_hbm.at[pl.ds(base, C)], ii_v)
                pltpu.sync_copy(jj_hbm.at[pl.ds(base, C)], jj_v)
                cq = pltpu.async_copy(q_hbm.at[ii_v], qrows, sem)
                ck = pltpu.async_copy(k_hbm.at[jj_v], krows, sem)
                cq.wait()
                ck.wait()

                lane = lax.iota(jnp.int32, L)

                def group_body(g, c2):
                    for p in range(L):
                        e = g * L + p
                        acc = qrows[e, pl.ds(0, L)] * krows[e, pl.ds(0, L)]
                        for s in range(1, F // L):
                            acc = acc + (qrows[e, pl.ds(s * L, L)] *
                                         krows[e, pl.ds(s * L, L)])
                        accflat[pl.ds(p * L, L)] = acc
                    outvec = plsc.load_gather(accflat, [lane * L])
                    for c in range(1, L):
                        outvec = outvec + plsc.load_gather(accflat, [lane * L + c])
                    out_v[pl.ds(g * L, L)] = outvec
                    return c2

                lax.fori_loop(0, C // L, group_body, 0)
                pltpu.sync_copy(out_v, out_hbm.at[pl.ds(base, C)])

            return carry

        lax.fori_loop(0, TPB, block_body, 0)

    return sc_kernel(q, k, idx_i, idx_j)


def kernel(x, idx_i, idx_j, Wq, bq, Wk, bk):
    scale = 1.0 / math.sqrt(F)
    W = jnp.concatenate([Wq, Wk], axis=1)
    b = jnp.concatenate([bq, bk])[None, :]
    q, k = _project(x, W, b, scale)
    return _sc_edge_dot(q, k, idx_i.astype(jnp.int32), idx_j.astype(jnp.int32))


# speedup vs baseline: 3.9433x; 1.5843x over previous
import functools
import math

import jax
import jax.numpy as jnp
from jax import lax
from jax.experimental import pallas as pl
from jax.experimental.pallas import tpu as pltpu
from jax.experimental.pallas import tpu_sc as plsc

N, F, E = 10000, 256, 160000
M_TILE = 400
C = 64
NBC = E // C
NC, NS, L = 2, 16, 16
NW = NC * NS
NB_LO = NBC // NW
EXTRA = NBC - NW * NB_LO
E_LO = NB_LO * C
E_HI = (NB_LO + 2) * C


def _proj_kernel(x_ref, w_ref, b_ref, q_ref, k_ref, *, scale):
    res = jnp.dot(x_ref[...], w_ref[...], preferred_element_type=jnp.float32)
    res = res + b_ref[...]
    q_ref[...] = res[:, :F] * scale
    k_ref[...] = res[:, F:]


def _project(x, W, b, scale):
    return pl.pallas_call(
        functools.partial(_proj_kernel, scale=scale),
        grid=(N // M_TILE,),
        in_specs=[
            pl.BlockSpec((M_TILE, F), lambda i: (i, 0)),
            pl.BlockSpec((F, 2 * F), lambda i: (0, 0)),
            pl.BlockSpec((1, 2 * F), lambda i: (0, 0)),
        ],
        out_specs=[
            pl.BlockSpec((M_TILE, F), lambda i: (i, 0)),
            pl.BlockSpec((M_TILE, F), lambda i: (i, 0)),
        ],
        out_shape=[
            jax.ShapeDtypeStruct((N, F), jnp.float32),
            jax.ShapeDtypeStruct((N, F), jnp.float32),
        ],
    )(x, W, b)


def _sc_edge_dot(q, k, idx_i, idx_j):
    mesh = plsc.VectorSubcoreMesh(core_axis_name="c", subcore_axis_name="s")

    @functools.partial(
        pl.kernel,
        mesh=mesh,
        out_type=jax.ShapeDtypeStruct((E,), jnp.float32),
        scratch_types=[
            pltpu.VMEM((E_HI,), jnp.int32),
            pltpu.VMEM((E_HI,), jnp.int32),
            pltpu.VMEM((C, F), jnp.float32),
            pltpu.VMEM((C, F), jnp.float32),
            pltpu.VMEM((C, F), jnp.float32),
            pltpu.VMEM((C, F), jnp.float32),
            pltpu.VMEM((E_HI,), jnp.float32),
            pltpu.VMEM((L * L,), jnp.float32),
            pltpu.SemaphoreType.DMA,
            pltpu.SemaphoreType.DMA,
        ],
        compiler_params=pltpu.CompilerParams(needs_layout_passes=False),
    )
    def sc_kernel(q_hbm, k_hbm, ii_hbm, jj_hbm, out_hbm,
                  ii_v, jj_v, qr0, kr0, qr1, kr1, out_v, accflat, semA, semB):
        wid = lax.axis_index("s") * NC + lax.axis_index("c")
        lt2 = jnp.minimum(wid, 2)
        nb = jnp.where(wid < 2, NB_LO + 2, NB_LO)
        ebase = (NB_LO * wid + 2 * lt2) * C

        pltpu.sync_copy(ii_hbm.at[pl.ds(ebase, E_LO)], ii_v.at[pl.ds(0, E_LO)])
        pltpu.sync_copy(jj_hbm.at[pl.ds(ebase, E_LO)], jj_v.at[pl.ds(0, E_LO)])

        @pl.when(wid < 2)
        def _():
            pltpu.sync_copy(ii_hbm.at[pl.ds(ebase + E_LO, E_HI - E_LO)],
                            ii_v.at[pl.ds(E_LO, E_HI - E_LO)])
            pltpu.sync_copy(jj_hbm.at[pl.ds(ebase + E_LO, E_HI - E_LO)],
                            jj_v.at[pl.ds(E_LO, E_HI - E_LO)])

        def issue(blk, qr, kr, sem):
            pltpu.async_copy(q_hbm.at[ii_v.at[pl.ds(blk * C, C)]], qr, sem)
            pltpu.async_copy(k_hbm.at[jj_v.at[pl.ds(blk * C, C)]], kr, sem)

        def drain(qr, kr, sem):
            pltpu.make_async_copy(q_hbm.at[pl.ds(0, C)], qr, sem).wait()
            pltpu.make_async_copy(k_hbm.at[pl.ds(0, C)], kr, sem).wait()

        lane = lax.iota(jnp.int32, L)

        def compute(blk, qr, kr):
            def group_body(g, c2):
                for p in range(L):
                    acc = qr[g * L + p, pl.ds(0, L)] * kr[g * L + p, pl.ds(0, L)]
                    for s in range(1, F // L):
                        acc = acc + (qr[g * L + p, pl.ds(s * L, L)] *
                                     kr[g * L + p, pl.ds(s * L, L)])
                    accflat[pl.ds(p * L, L)] = acc
                outvec = plsc.load_gather(accflat, [lane * L])
                for c in range(1, L):
                    outvec = outvec + plsc.load_gather(accflat, [lane * L + c])
                out_v[pl.ds(blk * C + g * L, L)] = outvec
                return c2

            lax.fori_loop(0, C // L, group_body, 0)

        issue(0, qr0, kr0, semA)

        def pair_body(i, carry):
            b0 = 2 * i
            issue(b0 + 1, qr1, kr1, semB)
            drain(qr0, kr0, semA)
            compute(b0, qr0, kr0)

            @pl.when(b0 + 2 < nb)
            def _():
                issue(b0 + 2, qr0, kr0, semA)

            drain(qr1, kr1, semB)
            compute(b0 + 1, qr1, kr1)
            return carry

        lax.fori_loop(0, nb // 2, pair_body, 0)

        pltpu.sync_copy(out_v.at[pl.ds(0, E_LO)], out_hbm.at[pl.ds(ebase, E_LO)])

        @pl.when(wid < 2)
        def _():
            pltpu.sync_copy(out_v.at[pl.ds(E_LO, E_HI - E_LO)],
                            out_hbm.at[pl.ds(ebase + E_LO, E_HI - E_LO)])

    return sc_kernel(q, k, idx_i, idx_j)


def kernel(x, idx_i, idx_j, Wq, bq, Wk, bk):
    scale = 1.0 / math.sqrt(F)
    W = jnp.concatenate([Wq, Wk], axis=1)
    b = jnp.concatenate([bq, bk])[None, :]
    q, k = _project(x, W, b, scale)
    return _sc_edge_dot(q, k, idx_i.astype(jnp.int32), idx_j.astype(jnp.int32))
